# Initial kernel scaffold; baseline (speedup 1.0000x reference)
import jax
import jax.numpy as jnp
from jax.experimental import pallas as pl
from jax.experimental.pallas import tpu as pltpu

_SORT8 = ((0, 1), (2, 3), (4, 5), (6, 7),
          (0, 2), (1, 3), (4, 6), (5, 7),
          (1, 2), (5, 6),
          (0, 4), (1, 5), (2, 6), (3, 7),
          (2, 4), (3, 5),
          (1, 2), (3, 4), (5, 6))

_BITONIC8 = ((0, 4), (1, 5), (2, 6), (3, 7),
             (0, 2), (1, 3), (4, 6), (5, 7),
             (0, 1), (2, 3), (4, 5), (6, 7))


def _ce(rows, i, j):
    hi = jnp.maximum(rows[i], rows[j])
    lo = jnp.minimum(rows[i], rows[j])
    rows[i], rows[j] = hi, lo


def _merge_fold(rows, h):
    b = [r[:h] for r in rows]
    c = [r[h:] for r in rows]
    rows = [jnp.maximum(b[r], c[7 - r]) for r in range(8)]
    for i, j in _BITONIC8:
        _ce(rows, i, j)
    return rows


def _topk_body(x_ref, o_ref):
    X = x_ref[0]
    rows = [X[r * 512:(r + 1) * 512, :] for r in range(8)]
    for i, j in _SORT8:
        _ce(rows, i, j)
    h = 256
    while h >= 1:
        rows = _merge_fold(rows, h)
        h //= 2
    rows = [r.reshape(8, 16) for r in rows]
    h = 4
    while h >= 1:
        rows = _merge_fold(rows, h)
        h //= 2
    o_ref[0] = jnp.concatenate(rows, axis=0)


def kernel(inputs):
    B, S, C = inputs.shape
    x4 = inputs.reshape(B, S * C // 128, 128)
    out = pl.pallas_call(
        _topk_body,
        grid=(B,),
        in_specs=[pl.BlockSpec((1, S * C // 128, 128), lambda b: (b, 0, 0))],
        out_specs=pl.BlockSpec((1, 8, C), lambda b: (b, 0, 0)),
        out_shape=jax.ShapeDtypeStruct((B, 8, C), jnp.float32),
        compiler_params=pltpu.CompilerParams(
            dimension_semantics=("parallel",),
        ),
    )(x4)
    return out


# speedup vs baseline: 45.8021x; 45.8021x over previous
import jax
import jax.numpy as jnp
from jax.experimental import pallas as pl
from jax.experimental.pallas import tpu as pltpu

_SORT8 = ((0, 1), (2, 3), (4, 5), (6, 7),
          (0, 2), (1, 3), (4, 6), (5, 7),
          (1, 2), (5, 6),
          (0, 4), (1, 5), (2, 6), (3, 7),
          (2, 4), (3, 5),
          (1, 2), (3, 4), (5, 6))

_BITONIC8 = ((0, 4), (1, 5), (2, 6), (3, 7),
             (0, 2), (1, 3), (4, 6), (5, 7),
             (0, 1), (2, 3), (4, 5), (6, 7))


def _ce(rows, i, j):
    hi = jnp.maximum(rows[i], rows[j])
    lo = jnp.minimum(rows[i], rows[j])
    rows[i], rows[j] = hi, lo


def _merge_fold(rows, h):
    b = [r[:h] for r in rows]
    c = [r[h:] for r in rows]
    rows = [jnp.maximum(b[r], c[7 - r]) for r in range(8)]
    for i, j in _BITONIC8:
        _ce(rows, i, j)
    return rows


def _topk_body(x_ref, o_ref):
    X = x_ref[0]
    rows = [X[r * 512:(r + 1) * 512, :] for r in range(8)]
    for i, j in _SORT8:
        _ce(rows, i, j)
    h = 256
    while h >= 1:
        rows = _merge_fold(rows, h)
        h //= 2
    w = 64
    while w >= 16:
        b = [r[:, :w] for r in rows]
        c = [r[:, w:] for r in rows]
        rows = [jnp.maximum(b[r], c[7 - r]) for r in range(8)]
        for i, j in _BITONIC8:
            _ce(rows, i, j)
        w //= 2
    o_ref[0] = jnp.concatenate(rows, axis=0)


def kernel(inputs):
    B, S, C = inputs.shape
    x4 = inputs.reshape(B, S * C // 128, 128)
    out = pl.pallas_call(
        _topk_body,
        grid=(B,),
        in_specs=[pl.BlockSpec((1, S * C // 128, 128), lambda b: (b, 0, 0))],
        out_specs=pl.BlockSpec((1, 8, C), lambda b: (b, 0, 0)),
        out_shape=jax.ShapeDtypeStruct((B, 8, C), jnp.float32),
        compiler_params=pltpu.CompilerParams(
            dimension_semantics=("parallel",),
        ),
    )(x4)
    return out


# speedup vs baseline: 48.3115x; 1.0548x over previous
import jax
import jax.numpy as jnp
from jax.experimental import pallas as pl
from jax.experimental.pallas import tpu as pltpu

_SORT8 = ((0, 1), (2, 3), (4, 5), (6, 7),
          (0, 2), (1, 3), (4, 6), (5, 7),
          (1, 2), (5, 6),
          (0, 4), (1, 5), (2, 6), (3, 7),
          (2, 4), (3, 5),
          (1, 2), (3, 4), (5, 6))

_BITONIC8 = ((0, 4), (1, 5), (2, 6), (3, 7),
             (0, 2), (1, 3), (4, 6), (5, 7),
             (0, 1), (2, 3), (4, 5), (6, 7))


def _ce(rows, i, j):
    hi = jnp.maximum(rows[i], rows[j])
    lo = jnp.minimum(rows[i], rows[j])
    rows[i], rows[j] = hi, lo


def _merge_fold(rows, h):
    b = [r[:h] for r in rows]
    c = [r[h:] for r in rows]
    rows = [jnp.maximum(b[r], c[7 - r]) for r in range(8)]
    for i, j in _BITONIC8:
        _ce(rows, i, j)
    return rows


def _sorted_chunk(X, base):
    rows = [X[base + r * 8:base + (r + 1) * 8, :] for r in range(8)]
    for i, j in _SORT8:
        _ce(rows, i, j)
    return rows


def _merge_into(acc, rows):
    acc = [jnp.maximum(acc[r], rows[7 - r]) for r in range(8)]
    for i, j in _BITONIC8:
        _ce(acc, i, j)
    return acc


def _topk_body(x_ref, o_ref):
    X = x_ref[0]
    acc0 = _sorted_chunk(X, 0)
    acc1 = _sorted_chunk(X, 64)
    for ci in range(1, 32):
        acc0 = _merge_into(acc0, _sorted_chunk(X, ci * 128))
        acc1 = _merge_into(acc1, _sorted_chunk(X, ci * 128 + 64))
    rows = _merge_into(acc0, acc1)
    h = 4
    while h >= 1:
        rows = _merge_fold(rows, h)
        h //= 2
    w = 64
    while w >= 16:
        b = [r[:, :w] for r in rows]
        c = [r[:, w:] for r in rows]
        rows = [jnp.maximum(b[r], c[7 - r]) for r in range(8)]
        for i, j in _BITONIC8:
            _ce(rows, i, j)
        w //= 2
    o_ref[0] = jnp.concatenate(rows, axis=0)


def kernel(inputs):
    B, S, C = inputs.shape
    x4 = inputs.reshape(B, S * C // 128, 128)
    out = pl.pallas_call(
        _topk_body,
        grid=(B,),
        in_specs=[pl.BlockSpec((1, S * C // 128, 128), lambda b: (b, 0, 0))],
        out_specs=pl.BlockSpec((1, 8, C), lambda b: (b, 0, 0)),
        out_shape=jax.ShapeDtypeStruct((B, 8, C), jnp.float32),
        compiler_params=pltpu.CompilerParams(
            dimension_semantics=("parallel",),
        ),
    )(x4)
    return out
